# baseline (speedup 1.0000x reference)
import functools

import jax
import jax.numpy as jnp
from jax import lax
from jax.experimental import pallas as pl
from jax.experimental.pallas import tpu as pltpu
from jax.experimental.pallas import tpu_sc as plsc

B = 8
N = 512
MV_OUT_CH = 10
EPS = 1e-05
SCALE = 1.0 / 8.0

_NC = 2
_NS = 16
_LANES = 16
_CHUNKS = N // _LANES


def _consts_body(wq_ref, wk_ref, wv_ref, wmv_ref, out_ref):
    wq = wq_ref[...]
    wk = wk_ref[...]
    wv = wv_ref[...]
    wmv = wmv_ref[...]

    qg = wq[1:2, :] + wq[16:17, :]
    k4 = wk[1:5, :]
    w4 = jnp.sum(k4 * qg, axis=1, keepdims=True) * SCALE
    lg = jnp.sum((wk[1:2, :] + wk[16:17, :]) * qg) * SCALE
    lb = jnp.sum(wk[4:5, :] * qg) * SCALE

    sel_r = lax.broadcasted_iota(jnp.int32, (160, MV_OUT_CH), 0)
    sel_c = lax.broadcasted_iota(jnp.int32, (160, MV_OUT_CH), 1)
    sel = (sel_r == sel_c * 16).astype(jnp.float32)
    wmv_sub = jnp.dot(wmv, sel, preferred_element_type=jnp.float32)

    u4 = jnp.dot(wv[1:5, :], wmv_sub, preferred_element_type=jnp.float32)
    ug = jnp.dot(wv[1:2, :] + wv[16:17, :], wmv_sub,
                 preferred_element_type=jnp.float32)
    ub = jnp.dot(wv[4:5, :], wmv_sub, preferred_element_type=jnp.float32)

    spread_r = lax.broadcasted_iota(jnp.int32, (MV_OUT_CH, 128), 0)
    spread_c = lax.broadcasted_iota(jnp.int32, (MV_OUT_CH, 128), 1)
    spread = (spread_r == spread_c).astype(jnp.float32)

    out_ref[...] = jnp.concatenate(
        [
            jnp.broadcast_to(w4, (4, 128)),
            jnp.broadcast_to(jnp.reshape(lg, (1, 1)), (1, 128)),
            jnp.broadcast_to(jnp.reshape(lb, (1, 1)), (1, 128)),
            jnp.dot(ug, spread, preferred_element_type=jnp.float32),
            jnp.dot(ub, spread, preferred_element_type=jnp.float32),
            jnp.dot(u4, spread, preferred_element_type=jnp.float32),
            jnp.zeros((4, 128), jnp.float32),
        ],
        axis=0,
    )


_consts_tc = pl.pallas_call(
    _consts_body,
    out_shape=jax.ShapeDtypeStruct((16, 128), jnp.float32),
)


def _sc_body(batch_hbm, consts_hbm, out_hbm, bv, cv, ov, sem0, sem1):
    wid = lax.axis_index("s")

    @pl.when(wid < B)
    def _():
        cp_b = pltpu.async_copy(batch_hbm.at[wid], bv, sem0)
        cp_c = pltpu.async_copy(consts_hbm, cv, sem1)
        cp_b.wait()
        cp_c.wait()

        w0 = cv[0, pl.ds(0, _LANES)]
        w1 = cv[1, pl.ds(0, _LANES)]
        w2 = cv[2, pl.ds(0, _LANES)]
        w3 = cv[3, pl.ds(0, _LANES)]
        lgv = cv[4, pl.ds(0, _LANES)]
        lbv = cv[5, pl.ds(0, _LANES)]
        ugv = cv[6, pl.ds(0, _LANES)]
        ubv = cv[7, pl.ds(0, _LANES)]
        u40 = cv[8, pl.ds(0, _LANES)]
        u41 = cv[9, pl.ds(0, _LANES)]
        u42 = cv[10, pl.ds(0, _LANES)]
        u43 = cv[11, pl.ds(0, _LANES)]

        neg_inf = jnp.full((_LANES,), -jnp.inf, jnp.float32)

        mx = neg_inf
        for i in range(_CHUNKS):
            b0 = bv[0, pl.ds(i * _LANES, _LANES)]
            b1 = bv[1, pl.ds(i * _LANES, _LANES)]
            b2 = bv[2, pl.ds(i * _LANES, _LANES)]
            b3 = bv[3, pl.ds(i * _LANES, _LANES)]
            l = b0 * w0 + b1 * w1 + b2 * w2 + b3 * w3
            valid = ((jnp.abs(b0) > EPS) & (jnp.abs(b1) > EPS)
                     & (jnp.abs(b2) > EPS) & (jnp.abs(b3) > EPS))
            mx = jnp.maximum(mx, jnp.where(valid, l, neg_inf))
        m = jnp.maximum(jnp.maximum(jnp.max(mx), jnp.max(lgv)), jnp.max(lbv))

        zero = jnp.zeros((_LANES,), jnp.float32)
        esum = zero
        s0 = zero
        s1 = zero
        s2 = zero
        s3 = zero
        for i in range(_CHUNKS):
            b0 = bv[0, pl.ds(i * _LANES, _LANES)]
            b1 = bv[1, pl.ds(i * _LANES, _LANES)]
            b2 = bv[2, pl.ds(i * _LANES, _LANES)]
            b3 = bv[3, pl.ds(i * _LANES, _LANES)]
            l = b0 * w0 + b1 * w1 + b2 * w2 + b3 * w3
            valid = ((jnp.abs(b0) > EPS) & (jnp.abs(b1) > EPS)
                     & (jnp.abs(b2) > EPS) & (jnp.abs(b3) > EPS))
            e = jnp.where(valid, jnp.exp(l - m), 0.0)
            esum = esum + e
            s0 = s0 + e * b0
            s1 = s1 + e * b1
            s2 = s2 + e * b2
            s3 = s3 + e * b3

        egv = jnp.exp(lgv - m)
        ebv = jnp.exp(lbv - m)
        etot = jnp.sum(esum) + jnp.max(egv) + jnp.max(ebv)
        outv = (egv * ugv + ebv * ubv
                + jnp.sum(s0) * u40 + jnp.sum(s1) * u41
                + jnp.sum(s2) * u42 + jnp.sum(s3) * u43) / etot
        ov[...] = outv
        pltpu.sync_copy(ov, out_hbm.at[wid])


@functools.cache
def _sc_main():
    mesh = plsc.VectorSubcoreMesh(
        core_axis_name="c", subcore_axis_name="s",
        num_cores=1, num_subcores=_NS,
    )
    return pl.kernel(
        _sc_body,
        out_type=jax.ShapeDtypeStruct((B, _LANES), jnp.float32),
        mesh=mesh,
        compiler_params=pltpu.CompilerParams(
            needs_layout_passes=False, skip_device_barrier=True),
        scratch_types=[
            pltpu.VMEM((4, N), jnp.float32),
            pltpu.VMEM((16, 128), jnp.float32),
            pltpu.VMEM((_LANES,), jnp.float32),
            pltpu.SemaphoreType.DMA,
            pltpu.SemaphoreType.DMA,
        ],
    )


@jax.jit
def kernel(batch, Wq, Wk, Wv, Wmv, Ws):
    del Ws
    consts = _consts_tc(Wq, Wk, Wv, Wmv)
    out2d = _sc_main()(batch, consts)
    return out2d[:, :MV_OUT_CH].reshape(B * MV_OUT_CH)


# speedup vs baseline: 1.0094x; 1.0094x over previous
import functools

import jax
import jax.numpy as jnp
import numpy as np
from jax import lax
from jax.experimental import pallas as pl
from jax.experimental.pallas import tpu as pltpu
from jax.experimental.pallas import tpu_sc as plsc

B = 8
N = 512
MV_OUT_CH = 10
EPS = 1e-05
SCALE = 1.0 / 8.0

_NS = 16
_LANES = 16
_CHUNKS = N // _LANES

_SEL = np.zeros((160, MV_OUT_CH), np.float32)
for _c in range(MV_OUT_CH):
    _SEL[16 * _c, _c] = 1.0
_S6 = np.zeros((6, 17), np.float32)
_S6[0, 1] = 1.0
_S6[0, 16] = 1.0
_S6[1, 4] = 1.0
for _c in range(4):
    _S6[2 + _c, 1 + _c] = 1.0
_SPREAD = np.zeros((MV_OUT_CH, 128), np.float32)
for _c in range(MV_OUT_CH):
    _SPREAD[_c, _c] = 1.0


def _consts_body(wq_ref, wk_ref, wv_ref, wmv_ref, sel_ref, s6_ref, spread_ref,
                 out_ref):
    wq = wq_ref[...]
    wk = wk_ref[...]
    wv = wv_ref[...]
    wmv = wmv_ref[...]

    qg = wq[1:2, :] + wq[16:17, :]
    kall = jnp.sum(wk * qg, axis=1, keepdims=True) * SCALE
    w4 = kall[1:5]
    lg = kall[1:2] + kall[16:17]
    lb = kall[4:5]

    wmv_sub = jnp.dot(wmv, sel_ref[...],
                      preferred_element_type=jnp.float32)
    t6 = jnp.dot(s6_ref[...], wv,
                 preferred_element_type=jnp.float32)
    u6 = jnp.dot(t6, wmv_sub, preferred_element_type=jnp.float32)
    rows6_11 = jnp.dot(u6, spread_ref[...],
                       preferred_element_type=jnp.float32)

    out_ref[...] = jnp.concatenate(
        [
            jnp.broadcast_to(w4, (4, 128)),
            jnp.broadcast_to(lg, (1, 128)),
            jnp.broadcast_to(lb, (1, 128)),
            rows6_11,
            jnp.zeros((4, 128), jnp.float32),
        ],
        axis=0,
    )


_consts_tc_call = pl.pallas_call(
    _consts_body,
    out_shape=jax.ShapeDtypeStruct((16, 128), jnp.float32),
)


def _consts_tc(wq, wk, wv, wmv):
    return _consts_tc_call(wq, wk, wv, wmv,
                           jnp.asarray(_SEL), jnp.asarray(_S6),
                           jnp.asarray(_SPREAD))


def _sc_body(batch_hbm, consts_hbm, out_hbm, bv, cv, ov, sem0, sem1):
    wid = lax.axis_index("s")

    @pl.when(wid < B)
    def _():
        cp_b = pltpu.async_copy(batch_hbm.at[wid], bv, sem0)
        cp_c = pltpu.async_copy(consts_hbm, cv, sem1)
        cp_b.wait()
        cp_c.wait()

        w0 = cv[0, pl.ds(0, _LANES)]
        w1 = cv[1, pl.ds(0, _LANES)]
        w2 = cv[2, pl.ds(0, _LANES)]
        w3 = cv[3, pl.ds(0, _LANES)]
        lgv = cv[4, pl.ds(0, _LANES)]
        lbv = cv[5, pl.ds(0, _LANES)]
        ugv = cv[6, pl.ds(0, _LANES)]
        ubv = cv[7, pl.ds(0, _LANES)]
        u40 = cv[8, pl.ds(0, _LANES)]
        u41 = cv[9, pl.ds(0, _LANES)]
        u42 = cv[10, pl.ds(0, _LANES)]
        u43 = cv[11, pl.ds(0, _LANES)]

        neg_inf = jnp.full((_LANES,), -jnp.inf, jnp.float32)

        mx = neg_inf
        for i in range(_CHUNKS):
            b0 = bv[0, pl.ds(i * _LANES, _LANES)]
            b1 = bv[1, pl.ds(i * _LANES, _LANES)]
            b2 = bv[2, pl.ds(i * _LANES, _LANES)]
            b3 = bv[3, pl.ds(i * _LANES, _LANES)]
            l = b0 * w0 + b1 * w1 + b2 * w2 + b3 * w3
            valid = ((jnp.abs(b0) > EPS) & (jnp.abs(b1) > EPS)
                     & (jnp.abs(b2) > EPS) & (jnp.abs(b3) > EPS))
            mx = jnp.maximum(mx, jnp.where(valid, l, neg_inf))
        m = jnp.maximum(jnp.maximum(jnp.max(mx), jnp.max(lgv)), jnp.max(lbv))

        zero = jnp.zeros((_LANES,), jnp.float32)
        esum = zero
        s0 = zero
        s1 = zero
        s2 = zero
        s3 = zero
        for i in range(_CHUNKS):
            b0 = bv[0, pl.ds(i * _LANES, _LANES)]
            b1 = bv[1, pl.ds(i * _LANES, _LANES)]
            b2 = bv[2, pl.ds(i * _LANES, _LANES)]
            b3 = bv[3, pl.ds(i * _LANES, _LANES)]
            l = b0 * w0 + b1 * w1 + b2 * w2 + b3 * w3
            valid = ((jnp.abs(b0) > EPS) & (jnp.abs(b1) > EPS)
                     & (jnp.abs(b2) > EPS) & (jnp.abs(b3) > EPS))
            e = jnp.where(valid, jnp.exp(l - m), 0.0)
            esum = esum + e
            s0 = s0 + e * b0
            s1 = s1 + e * b1
            s2 = s2 + e * b2
            s3 = s3 + e * b3

        egv = jnp.exp(lgv - m)
        ebv = jnp.exp(lbv - m)
        etot = jnp.sum(esum) + jnp.max(egv) + jnp.max(ebv)
        outv = (egv * ugv + ebv * ubv
                + jnp.sum(s0) * u40 + jnp.sum(s1) * u41
                + jnp.sum(s2) * u42 + jnp.sum(s3) * u43) / etot
        ov[...] = outv
        pltpu.sync_copy(ov, out_hbm.at[wid])


@functools.cache
def _sc_main():
    mesh = plsc.VectorSubcoreMesh(
        core_axis_name="c", subcore_axis_name="s",
        num_cores=1, num_subcores=_NS,
    )
    return pl.kernel(
        _sc_body,
        out_type=jax.ShapeDtypeStruct((B, _LANES), jnp.float32),
        mesh=mesh,
        compiler_params=pltpu.CompilerParams(needs_layout_passes=False),
        scratch_types=[
            pltpu.VMEM((4, N), jnp.float32),
            pltpu.VMEM((16, 128), jnp.float32),
            pltpu.VMEM((_LANES,), jnp.float32),
            pltpu.SemaphoreType.DMA,
            pltpu.SemaphoreType.DMA,
        ],
    )


@jax.jit
def kernel(batch, Wq, Wk, Wv, Wmv, Ws):
    del Ws
    consts = _consts_tc(Wq, Wk, Wv, Wmv)
    out2d = _sc_main()(batch, consts)
    return out2d[:, :MV_OUT_CH].reshape(B * MV_OUT_CH)
